# baseline (speedup 1.0000x reference)
import functools

import jax
import jax.numpy as jnp
from jax import lax
from jax.experimental import pallas as pl
from jax.experimental.pallas import tpu as pltpu
from jax.experimental.pallas import tpu_sc as plsc

N = 16384
K = 8192
D = 64
BETA = 0.25
TN = 2048
KT = 4096
NB = N // TN
NKC = K // KT
INT_MAX = 2**31 - 1


def _assign_body(zs_ref, cb_ref, z2_ref, c2_ref, idx_ref, vq_ref, com_ref,
                 perp_ref, uniq_ref, counts, lsum):
    i = pl.program_id(0)

    @pl.when(i == 0)
    def _init():
        counts[...] = jnp.zeros_like(counts)
        lsum[0] = jnp.float32(0.0)

    zs = zs_ref[...]
    z2 = z2_ref[...]

    halves = []
    for h in range(2):
        cb = cb_ref[pl.ds(h * KT, KT), :]
        c2 = c2_ref[0, pl.ds(h * KT, KT)]
        dotn = lax.dot_general(zs, cb, (((1,), (1,)), ((), ())),
                               preferred_element_type=jnp.float32)
        d2 = (z2 + dotn) + c2[None, :]
        dist = jnp.sqrt(jnp.maximum(d2, 0.0))
        mval = jnp.min(dist, axis=1)
        iota = lax.broadcasted_iota(jnp.int32, (TN, KT), 1) + (h * KT)
        cand = jnp.where(dist == mval[:, None], iota, INT_MAX)
        lidx = jnp.min(cand, axis=1)
        halves.append((mval, lidx))
    (u1, i1), (u2, i2) = halves
    u1b = u1.astype(jnp.bfloat16).astype(jnp.float32)
    take2 = u2 < u1b
    aidx = jnp.where(take2, i2, i1)
    winval = jnp.where(take2, u2, u1)

    idx_ref[...] = aidx
    lsum[0] += jnp.sum(winval * winval)

    ones_row = jnp.ones((1, TN), jnp.bfloat16)
    for kc in range(K // KT):
        iota = lax.broadcasted_iota(jnp.int32, (TN, KT), 1) + (kc * KT)
        oh = jnp.where(iota == aidx[:, None], 1.0, 0.0).astype(jnp.bfloat16)
        colsum = lax.dot_general(ones_row, oh, (((1,), (0,)), ((), ())),
                                 preferred_element_type=jnp.float32)
        counts[pl.ds(kc * KT, KT)] += colsum[0]

    @pl.when(i == NB - 1)
    def _finalize():
        vq = lsum[0] * (1.0 / (N * D))
        vq_ref[...] = jnp.full((1, 1), vq, jnp.float32)
        com_ref[...] = jnp.full((1, 1), BETA * vq, jnp.float32)
        probs = counts[...] * (1.0 / N)
        ent = -jnp.sum(probs * jnp.log(probs + 1e-10))
        perp_ref[...] = jnp.full((1, 1), jnp.exp(ent), jnp.float32)
        uniq = jnp.sum((counts[...] > 0).astype(jnp.int32))
        uniq_ref[...] = jnp.full((1, 1), uniq, jnp.int32)


_assign = pl.pallas_call(
    _assign_body,
    grid=(NB,),
    in_specs=[
        pl.BlockSpec((TN, D), lambda i: (i, 0)),
        pl.BlockSpec((K, D), lambda i: (0, 0)),
        pl.BlockSpec((TN, 1), lambda i: (i, 0)),
        pl.BlockSpec((1, K), lambda i: (0, 0)),
    ],
    out_specs=[
        pl.BlockSpec((TN,), lambda i: (i,)),
        pl.BlockSpec((1, 1), lambda i: (0, 0)),
        pl.BlockSpec((1, 1), lambda i: (0, 0)),
        pl.BlockSpec((1, 1), lambda i: (0, 0)),
        pl.BlockSpec((1, 1), lambda i: (0, 0)),
    ],
    out_shape=[
        jax.ShapeDtypeStruct((N,), jnp.int32),
        jax.ShapeDtypeStruct((1, 1), jnp.float32),
        jax.ShapeDtypeStruct((1, 1), jnp.float32),
        jax.ShapeDtypeStruct((1, 1), jnp.float32),
        jax.ShapeDtypeStruct((1, 1), jnp.int32),
    ],
    scratch_shapes=[
        pltpu.VMEM((K,), jnp.float32),
        pltpu.SMEM((1,), jnp.float32),
    ],
    compiler_params=pltpu.CompilerParams(
        dimension_semantics=("arbitrary",)),
)


@functools.cache
def _sc_gather():
    info = plsc.get_sparse_core_info()
    nw = info.num_cores * info.num_subcores
    bpw = N // nw

    @functools.partial(
        pl.kernel,
        mesh=plsc.VectorSubcoreMesh(core_axis_name="c", subcore_axis_name="s"),
        out_type=jax.ShapeDtypeStruct((N, D), jnp.float32),
        scratch_types=[
            pltpu.VMEM((bpw,), jnp.int32),
            pltpu.VMEM((bpw, D), jnp.float32),
            pltpu.SemaphoreType.DMA,
        ],
        compiler_params=pltpu.CompilerParams(use_tc_tiling_on_sc=False),
    )
    def gather(cb_hbm, idx_hbm, out_hbm, idx_v, rows_v, sem):
        wid = lax.axis_index("s") * info.num_cores + lax.axis_index("c")
        base = wid * bpw
        pltpu.sync_copy(idx_hbm.at[pl.ds(base, bpw)], idx_v)
        pltpu.async_copy(cb_hbm.at[idx_v], rows_v, sem).wait()
        pltpu.sync_copy(rows_v, out_hbm.at[pl.ds(base, bpw)])

    return gather


def kernel(z_feats, batch_ids, codebook):
    del batch_ids
    z2 = jnp.sum(z_feats ** 2, axis=1, keepdims=True)
    c2 = jnp.sum(codebook ** 2, axis=1)[None, :]
    zs = (-2.0 * z_feats).astype(jnp.bfloat16)
    cb16 = codebook.astype(jnp.bfloat16)
    idx, vq, com, perp, uniq = _assign(zs, cb16, z2, c2)
    quantized = _sc_gather()(codebook, idx)
    return (quantized, vq.reshape(()), com.reshape(()), idx,
            perp.reshape(()), uniq.reshape(()))


# speedup vs baseline: 1.2215x; 1.2215x over previous
import functools

import jax
import jax.numpy as jnp
from jax import lax
from jax.experimental import pallas as pl
from jax.experimental.pallas import tpu as pltpu
from jax.experimental.pallas import tpu_sc as plsc

N = 16384
K = 8192
D = 64
BETA = 0.25
TN = 1024
KT = 4096
NB = N // TN
NKC = K // KT
INT_MAX = 2**31 - 1


def _assign_body(zs_ref, cb_ref, z2_ref, c2_ref, idx_ref, vq_ref, com_ref,
                 perp_ref, uniq_ref, counts, lsum):
    i = pl.program_id(0)

    @pl.when(i == 0)
    def _init():
        counts[...] = jnp.zeros_like(counts)
        lsum[0] = jnp.float32(0.0)

    zs = zs_ref[...]
    z2 = z2_ref[...]

    halves = []
    for h in range(2):
        cb = cb_ref[pl.ds(h * KT, KT), :]
        c2 = c2_ref[0, pl.ds(h * KT, KT)]
        dotn = lax.dot_general(zs, cb, (((1,), (1,)), ((), ())),
                               preferred_element_type=jnp.float32)
        d2 = (z2 + dotn) + c2[None, :]
        dist = jnp.sqrt(jnp.maximum(d2, 0.0))
        mval = jnp.min(dist, axis=1)
        iota = lax.broadcasted_iota(jnp.int32, (TN, KT), 1) + (h * KT)
        cand = jnp.where(dist == mval[:, None], iota, INT_MAX)
        lidx = jnp.min(cand, axis=1)
        halves.append((mval, lidx))
    (u1, i1), (u2, i2) = halves
    u1b = u1.astype(jnp.bfloat16).astype(jnp.float32)
    take2 = u2 < u1b
    aidx = jnp.where(take2, i2, i1)
    winval = jnp.where(take2, u2, u1)

    idx_ref[...] = aidx
    lsum[0] += jnp.sum(winval * winval)

    ones_row = jnp.ones((1, TN), jnp.bfloat16)
    for kc in range(K // KT):
        iota = lax.broadcasted_iota(jnp.int32, (TN, KT), 1) + (kc * KT)
        oh = jnp.where(iota == aidx[:, None], 1.0, 0.0).astype(jnp.bfloat16)
        colsum = lax.dot_general(ones_row, oh, (((1,), (0,)), ((), ())),
                                 preferred_element_type=jnp.float32)
        counts[pl.ds(kc * KT, KT)] += colsum[0]

    @pl.when(i == NB - 1)
    def _finalize():
        vq = lsum[0] * (1.0 / (N * D))
        vq_ref[...] = jnp.full((1, 1), vq, jnp.float32)
        com_ref[...] = jnp.full((1, 1), BETA * vq, jnp.float32)
        probs = counts[...] * (1.0 / N)
        ent = -jnp.sum(probs * jnp.log(probs + 1e-10))
        perp_ref[...] = jnp.full((1, 1), jnp.exp(ent), jnp.float32)
        uniq = jnp.sum((counts[...] > 0).astype(jnp.int32))
        uniq_ref[...] = jnp.full((1, 1), uniq, jnp.int32)


_assign = pl.pallas_call(
    _assign_body,
    grid=(NB,),
    in_specs=[
        pl.BlockSpec((TN, D), lambda i: (i, 0)),
        pl.BlockSpec((K, D), lambda i: (0, 0)),
        pl.BlockSpec((TN, 1), lambda i: (i, 0)),
        pl.BlockSpec((1, K), lambda i: (0, 0)),
    ],
    out_specs=[
        pl.BlockSpec((TN,), lambda i: (i,)),
        pl.BlockSpec((1, 1), lambda i: (0, 0)),
        pl.BlockSpec((1, 1), lambda i: (0, 0)),
        pl.BlockSpec((1, 1), lambda i: (0, 0)),
        pl.BlockSpec((1, 1), lambda i: (0, 0)),
    ],
    out_shape=[
        jax.ShapeDtypeStruct((N,), jnp.int32),
        jax.ShapeDtypeStruct((1, 1), jnp.float32),
        jax.ShapeDtypeStruct((1, 1), jnp.float32),
        jax.ShapeDtypeStruct((1, 1), jnp.float32),
        jax.ShapeDtypeStruct((1, 1), jnp.int32),
    ],
    scratch_shapes=[
        pltpu.VMEM((K,), jnp.float32),
        pltpu.SMEM((1,), jnp.float32),
    ],
    compiler_params=pltpu.CompilerParams(
        dimension_semantics=("arbitrary",)),
)


@functools.cache
def _sc_gather():
    info = plsc.get_sparse_core_info()
    nw = info.num_cores * info.num_subcores
    bpw = N // nw

    @functools.partial(
        pl.kernel,
        mesh=plsc.VectorSubcoreMesh(core_axis_name="c", subcore_axis_name="s"),
        out_type=jax.ShapeDtypeStruct((N, D), jnp.float32),
        scratch_types=[
            pltpu.VMEM((bpw,), jnp.int32),
            pltpu.VMEM((bpw, D), jnp.float32),
            pltpu.SemaphoreType.DMA,
        ],
        compiler_params=pltpu.CompilerParams(use_tc_tiling_on_sc=False),
    )
    def gather(cb_hbm, idx_hbm, out_hbm, idx_v, rows_v, sem):
        wid = lax.axis_index("s") * info.num_cores + lax.axis_index("c")
        base = wid * bpw
        pltpu.sync_copy(idx_hbm.at[pl.ds(base, bpw)], idx_v)
        pltpu.async_copy(cb_hbm.at[idx_v], rows_v, sem).wait()
        pltpu.sync_copy(rows_v, out_hbm.at[pl.ds(base, bpw)])

    return gather


def kernel(z_feats, batch_ids, codebook):
    del batch_ids
    z2 = jnp.sum(z_feats ** 2, axis=1, keepdims=True)
    c2 = jnp.sum(codebook ** 2, axis=1)[None, :]
    zs = (-2.0 * z_feats).astype(jnp.bfloat16)
    cb16 = codebook.astype(jnp.bfloat16)
    idx, vq, com, perp, uniq = _assign(zs, cb16, z2, c2)
    quantized = _sc_gather()(codebook, idx)
    return (quantized, vq.reshape(()), com.reshape(()), idx,
            perp.reshape(()), uniq.reshape(()))


# speedup vs baseline: 1.2958x; 1.0608x over previous
import functools

import jax
import jax.numpy as jnp
from jax import lax
from jax.experimental import pallas as pl
from jax.experimental.pallas import tpu as pltpu
from jax.experimental.pallas import tpu_sc as plsc

N = 16384
K = 8192
D = 64
BETA = 0.25
TN = 1024
KT = 4096
NB = N // TN
NKC = K // KT
INT_MAX = 2**31 - 1


def _assign_body(zs_ref, cb_ref, z2_ref, c2_ref, idx_ref, vq_ref, com_ref,
                 perp_ref, uniq_ref, counts, lsum):
    i = pl.program_id(0)

    @pl.when(i == 0)
    def _init():
        counts[...] = jnp.zeros_like(counts)
        lsum[0] = jnp.float32(0.0)

    zs = zs_ref[...]
    z2 = z2_ref[...]

    halves = []
    for h in range(2):
        cb = cb_ref[pl.ds(h * KT, KT), :]
        c2 = c2_ref[0, pl.ds(h * KT, KT)]
        dotn = lax.dot_general(zs, cb, (((1,), (1,)), ((), ())),
                               preferred_element_type=jnp.float32)
        d2 = (z2 + dotn) + c2[None, :]
        dist = jnp.sqrt(jnp.maximum(d2, 0.0))
        mval = jnp.min(dist, axis=1)
        iota = lax.broadcasted_iota(
            jnp.int32, (TN, KT), 1).astype(jnp.float32)
        cand = jnp.where(dist == mval[:, None], iota, jnp.float32(KT))
        lidx = jnp.min(cand, axis=1).astype(jnp.int32) + (h * KT)
        halves.append((mval, lidx))
    (u1, i1), (u2, i2) = halves
    u1b = u1.astype(jnp.bfloat16).astype(jnp.float32)
    take2 = u2 < u1b
    aidx = jnp.where(take2, i2, i1)
    winval = jnp.where(take2, u2, u1)

    idx_ref[...] = aidx
    lsum[0] += jnp.sum(winval * winval)

    ones_row = jnp.ones((1, TN), jnp.bfloat16)
    aidx_f = aidx.astype(jnp.float32)
    iota = lax.broadcasted_iota(jnp.int32, (TN, KT), 1).astype(jnp.float32)
    for kc in range(K // KT):
        local = aidx_f - float(kc * KT)
        oh = jnp.where(iota == local[:, None], 1.0, 0.0).astype(jnp.bfloat16)
        colsum = lax.dot_general(ones_row, oh, (((1,), (0,)), ((), ())),
                                 preferred_element_type=jnp.float32)
        counts[pl.ds(kc * KT, KT)] += colsum[0]

    @pl.when(i == NB - 1)
    def _finalize():
        vq = lsum[0] * (1.0 / (N * D))
        vq_ref[...] = jnp.full((1, 1), vq, jnp.float32)
        com_ref[...] = jnp.full((1, 1), BETA * vq, jnp.float32)
        probs = counts[...] * (1.0 / N)
        ent = -jnp.sum(probs * jnp.log(probs + 1e-10))
        perp_ref[...] = jnp.full((1, 1), jnp.exp(ent), jnp.float32)
        uniq = jnp.sum((counts[...] > 0).astype(jnp.int32))
        uniq_ref[...] = jnp.full((1, 1), uniq, jnp.int32)


_assign = pl.pallas_call(
    _assign_body,
    grid=(NB,),
    in_specs=[
        pl.BlockSpec((TN, D), lambda i: (i, 0)),
        pl.BlockSpec((K, D), lambda i: (0, 0)),
        pl.BlockSpec((TN, 1), lambda i: (i, 0)),
        pl.BlockSpec((1, K), lambda i: (0, 0)),
    ],
    out_specs=[
        pl.BlockSpec((TN,), lambda i: (i,)),
        pl.BlockSpec((1, 1), lambda i: (0, 0)),
        pl.BlockSpec((1, 1), lambda i: (0, 0)),
        pl.BlockSpec((1, 1), lambda i: (0, 0)),
        pl.BlockSpec((1, 1), lambda i: (0, 0)),
    ],
    out_shape=[
        jax.ShapeDtypeStruct((N,), jnp.int32),
        jax.ShapeDtypeStruct((1, 1), jnp.float32),
        jax.ShapeDtypeStruct((1, 1), jnp.float32),
        jax.ShapeDtypeStruct((1, 1), jnp.float32),
        jax.ShapeDtypeStruct((1, 1), jnp.int32),
    ],
    scratch_shapes=[
        pltpu.VMEM((K,), jnp.float32),
        pltpu.SMEM((1,), jnp.float32),
    ],
    compiler_params=pltpu.CompilerParams(
        dimension_semantics=("arbitrary",)),
)


@functools.cache
def _sc_gather():
    info = plsc.get_sparse_core_info()
    nw = info.num_cores * info.num_subcores
    bpw = N // nw

    @functools.partial(
        pl.kernel,
        mesh=plsc.VectorSubcoreMesh(core_axis_name="c", subcore_axis_name="s"),
        out_type=jax.ShapeDtypeStruct((N, D), jnp.float32),
        scratch_types=[
            pltpu.VMEM((bpw,), jnp.int32),
            pltpu.VMEM((bpw, D), jnp.float32),
            pltpu.SemaphoreType.DMA,
        ],
        compiler_params=pltpu.CompilerParams(use_tc_tiling_on_sc=False),
    )
    def gather(cb_hbm, idx_hbm, out_hbm, idx_v, rows_v, sem):
        wid = lax.axis_index("s") * info.num_cores + lax.axis_index("c")
        base = wid * bpw
        pltpu.sync_copy(idx_hbm.at[pl.ds(base, bpw)], idx_v)
        pltpu.async_copy(cb_hbm.at[idx_v], rows_v, sem).wait()
        pltpu.sync_copy(rows_v, out_hbm.at[pl.ds(base, bpw)])

    return gather


def kernel(z_feats, batch_ids, codebook):
    del batch_ids
    z2 = jnp.sum(z_feats ** 2, axis=1, keepdims=True)
    c2 = jnp.sum(codebook ** 2, axis=1)[None, :]
    zs = (-2.0 * z_feats).astype(jnp.bfloat16)
    cb16 = codebook.astype(jnp.bfloat16)
    idx, vq, com, perp, uniq = _assign(zs, cb16, z2, c2)
    quantized = _sc_gather()(codebook, idx)
    return (quantized, vq.reshape(()), com.reshape(()), idx,
            perp.reshape(()), uniq.reshape(()))


# speedup vs baseline: 1.2963x; 1.0004x over previous
import functools

import jax
import jax.numpy as jnp
from jax import lax
from jax.experimental import pallas as pl
from jax.experimental.pallas import tpu as pltpu
from jax.experimental.pallas import tpu_sc as plsc

N = 16384
K = 8192
D = 64
BETA = 0.25
TN = 1024
KT = 4096
NB = N // TN


def _assign_body(zs_ref, cb_ref, z2_ref, c2_ref, idx_ref, vq_ref, com_ref,
                 perp_ref, uniq_ref, counts, lsum):
    i = pl.program_id(0)

    @pl.when(i == 0)
    def _init():
        counts[...] = jnp.zeros_like(counts)
        lsum[0] = jnp.float32(0.0)

    zs = zs_ref[...]
    z2 = z2_ref[...]

    halves = []
    for h in range(2):
        cb = cb_ref[pl.ds(h * KT, KT), :]
        c2 = c2_ref[0, pl.ds(h * KT, KT)]
        dotn = lax.dot_general(zs, cb, (((1,), (1,)), ((), ())),
                               preferred_element_type=jnp.float32)
        d2 = (z2 + dotn) + c2[None, :]
        dist = jnp.sqrt(jnp.maximum(d2, 0.0))
        mval = jnp.min(dist, axis=1)
        iota = lax.broadcasted_iota(
            jnp.int32, (TN, KT), 1).astype(jnp.float32)
        cand = jnp.where(dist == mval[:, None], iota, jnp.float32(KT))
        lidx = jnp.min(cand, axis=1).astype(jnp.int32) + (h * KT)
        halves.append((mval, lidx))
    (u1, i1), (u2, i2) = halves
    u1b = u1.astype(jnp.bfloat16).astype(jnp.float32)
    take2 = u2 < u1b
    aidx = jnp.where(take2, i2, i1)
    winval = jnp.where(take2, u2, u1)

    idx_ref[...] = aidx
    lsum[0] += jnp.sum(winval * winval)

    ones_row = jnp.ones((1, TN), jnp.bfloat16)
    aidx_f = aidx.astype(jnp.float32)
    iota = lax.broadcasted_iota(jnp.int32, (TN, KT), 1).astype(jnp.float32)
    for kc in range(K // KT):
        local = aidx_f - float(kc * KT)
        oh = jnp.where(iota == local[:, None], 1.0, 0.0).astype(jnp.bfloat16)
        colsum = lax.dot_general(ones_row, oh, (((1,), (0,)), ((), ())),
                                 preferred_element_type=jnp.float32)
        counts[pl.ds(kc * KT, KT)] += colsum[0]

    @pl.when(i == NB - 1)
    def _finalize():
        vq = lsum[0] * (1.0 / (N * D))
        vq_ref[...] = jnp.full((1, 1), vq, jnp.float32)
        com_ref[...] = jnp.full((1, 1), BETA * vq, jnp.float32)
        probs = counts[...] * (1.0 / N)
        ent = -jnp.sum(probs * jnp.log(probs + 1e-10))
        perp_ref[...] = jnp.full((1, 1), jnp.exp(ent), jnp.float32)
        uniq = jnp.sum((counts[...] > 0).astype(jnp.int32))
        uniq_ref[...] = jnp.full((1, 1), uniq, jnp.int32)


_assign = pl.pallas_call(
    _assign_body,
    grid=(NB,),
    in_specs=[
        pl.BlockSpec((TN, D), lambda i: (i, 0)),
        pl.BlockSpec((K, D), lambda i: (0, 0)),
        pl.BlockSpec((TN, 1), lambda i: (i, 0)),
        pl.BlockSpec((1, K), lambda i: (0, 0)),
    ],
    out_specs=[
        pl.BlockSpec((TN,), lambda i: (i,)),
        pl.BlockSpec((1, 1), lambda i: (0, 0)),
        pl.BlockSpec((1, 1), lambda i: (0, 0)),
        pl.BlockSpec((1, 1), lambda i: (0, 0)),
        pl.BlockSpec((1, 1), lambda i: (0, 0)),
    ],
    out_shape=[
        jax.ShapeDtypeStruct((N,), jnp.int32),
        jax.ShapeDtypeStruct((1, 1), jnp.float32),
        jax.ShapeDtypeStruct((1, 1), jnp.float32),
        jax.ShapeDtypeStruct((1, 1), jnp.float32),
        jax.ShapeDtypeStruct((1, 1), jnp.int32),
    ],
    scratch_shapes=[
        pltpu.VMEM((K,), jnp.float32),
        pltpu.SMEM((1,), jnp.float32),
    ],
    compiler_params=pltpu.CompilerParams(
        dimension_semantics=("arbitrary",)),
)


@functools.cache
def _sc_gather():
    info = plsc.get_sparse_core_info()
    nw = info.num_cores * info.num_subcores
    bpw = N // nw

    @functools.partial(
        pl.kernel,
        mesh=plsc.VectorSubcoreMesh(core_axis_name="c", subcore_axis_name="s"),
        out_type=jax.ShapeDtypeStruct((N, D), jnp.float32),
        scratch_types=[
            pltpu.VMEM((bpw,), jnp.int32),
            pltpu.VMEM((bpw, D), jnp.float32),
            pltpu.SemaphoreType.DMA,
        ],
        compiler_params=pltpu.CompilerParams(use_tc_tiling_on_sc=False),
    )
    def gather(cb_hbm, idx_hbm, out_hbm, idx_v, rows_v, sem):
        wid = lax.axis_index("s") * info.num_cores + lax.axis_index("c")
        base = wid * bpw
        pltpu.sync_copy(idx_hbm.at[pl.ds(base, bpw)], idx_v)
        pltpu.async_copy(cb_hbm.at[idx_v], rows_v, sem).wait()
        pltpu.sync_copy(rows_v, out_hbm.at[pl.ds(base, bpw)])

    return gather


def kernel(z_feats, batch_ids, codebook):
    del batch_ids
    z2 = jnp.sum(z_feats ** 2, axis=1, keepdims=True)
    c2 = jnp.sum(codebook ** 2, axis=1)[None, :]
    zs = (-2.0 * z_feats).astype(jnp.bfloat16)
    cb16 = codebook.astype(jnp.bfloat16)
    idx, vq, com, perp, uniq = _assign(zs, cb16, z2, c2)
    quantized = _sc_gather()(codebook, idx)
    return (quantized, vq.reshape(()), com.reshape(()), idx,
            perp.reshape(()), uniq.reshape(()))
